# baseline (speedup 1.0000x reference)
import jax
import jax.numpy as jnp
from jax import lax
from jax.experimental import pallas as pl

NUM_CLASSES_K = 128
BATCH = 64
PIX = 32 * 32


def _onehot_body(x_ref, o_ref):
    x = x_ref[...]
    cls = lax.broadcasted_iota(jnp.int32, o_ref.shape, 1)
    o_ref[...] = (x[:, None, :] == cls).astype(jnp.float32)


def kernel(x):
    x = x.astype(jnp.int32).reshape(BATCH, PIX)
    bb = 8
    out = pl.pallas_call(
        _onehot_body,
        grid=(BATCH // bb,),
        in_specs=[pl.BlockSpec((bb, PIX), lambda i: (i, 0))],
        out_specs=pl.BlockSpec((bb, NUM_CLASSES_K, PIX), lambda i: (i, 0, 0)),
        out_shape=jax.ShapeDtypeStruct((BATCH, NUM_CLASSES_K, PIX), jnp.float32),
    )(x)
    return out.reshape(BATCH, NUM_CLASSES_K, 32, 32)
